# baseline (speedup 1.0000x reference)
import functools

import jax
import jax.numpy as jnp
from jax import lax
from jax.experimental import pallas as pl
from jax.experimental.pallas import tpu as pltpu
from jax.experimental.pallas import tpu_sc as plsc

N = 10000
E = 320000
NT = 16
NC = 2
CH = 128
NCHB = (E // CH) // NT
XTRA = (E // CH) - NCHB * NT
AGG_ROWS = 10000
DEG_ROWS = 10240
RPT = 640
RPT_LAST = N - (NT - 1) * RPT
R = 2000
NBLK = (2 * N) // R

@functools.cache
def _get_mesh():
    return plsc.VectorSubcoreMesh(core_axis_name="c", subcore_axis_name="s",
                                  num_cores=NC, num_subcores=NT)


def _deg_body(ei1_hbm, ei2_hbm, out_hbm, ones_v, didx, acc_sh, sem_i):
    cid = lax.axis_index("c")
    sid = lax.axis_index("s")
    nch = NCHB + jnp.where(sid < XTRA, 1, 0)
    base = sid * NCHB + jnp.minimum(sid, XTRA)

    def _fill(i, carry):
        ones_v[i] = jnp.ones((16,), jnp.float32)
        return carry

    lax.fori_loop(0, CH, _fill, 0)
    for k in range(DEG_ROWS // NT // CH):
        pltpu.sync_copy(ones_v, acc_sh.at[pl.ds(sid * (DEG_ROWS // NT) + k * CH, CH)])

    def _load_idx(c, slot):
        @pl.when(cid == 0)
        def _():
            pltpu.async_copy(ei1_hbm.at[1, pl.ds((base + c) * CH, CH)],
                             didx.at[slot], sem_i.at[slot])

        @pl.when(cid == 1)
        def _():
            pltpu.async_copy(ei2_hbm.at[1, pl.ds((base + c) * CH, CH)],
                             didx.at[slot], sem_i.at[slot])

    def _wait_idx(c, slot):
        pltpu.make_async_copy(ei1_hbm.at[1, pl.ds(0, CH)], didx.at[slot],
                              sem_i.at[slot]).wait()

    _load_idx(0, 0)
    _load_idx(1, 1)
    plsc.subcore_barrier()

    def _chunk(c, carry):
        @pl.when(c + 2 < nch)
        def _():
            _load_idx(c + 2, lax.rem(c + 2, 4))

        _wait_idx(c, lax.rem(c, 4))
        pltpu.sync_copy(ones_v, acc_sh.at[didx.at[lax.rem(c, 4)]], add=True)
        return carry

    lax.fori_loop(0, nch, _chunk, 0)
    plsc.subcore_barrier()

@functools.cache
def _deg_kernel_fn():
    return pl.kernel(
        _deg_body,
        out_type=jax.ShapeDtypeStruct((2 * N, 16), jnp.float32),
        mesh=_get_mesh(),
        scratch_types=[
            pltpu.VMEM((CH, 16), jnp.float32),
            pltpu.VMEM((4, CH), jnp.int32),
            pltpu.VMEM_SHARED((DEG_ROWS, 16), jnp.float32),
            pltpu.SemaphoreType.DMA((4,)),
        ],
        compiler_params=pltpu.CompilerParams(use_tc_tiling_on_sc=False),
    )


def _deg_kernel(ei1, ei2):
    return _deg_kernel_fn()(ei1, ei2)


def _agg_body(h_hbm, ei1_hbm, ei2_hbm, out_hbm,
              sidx, didx, rows_v, acc_sh, sem_i, sem_g, sem_s):
    cid = lax.axis_index("c")
    sid = lax.axis_index("s")
    nch = NCHB + jnp.where(sid < XTRA, 1, 0)
    base = sid * NCHB + jnp.minimum(sid, XTRA)

    @pl.when(sid < NT - 1)
    def _():
        pltpu.sync_copy(h_hbm.at[pl.ds(cid * N + sid * RPT, RPT)],
                        acc_sh.at[pl.ds(sid * RPT, RPT)])

    @pl.when(sid == NT - 1)
    def _():
        pltpu.sync_copy(h_hbm.at[pl.ds(cid * N + (NT - 1) * RPT, RPT_LAST)],
                        acc_sh.at[pl.ds((NT - 1) * RPT, RPT_LAST)])

    def _load_idx(c, slot):
        @pl.when(cid == 0)
        def _():
            pltpu.async_copy(ei1_hbm.at[0, pl.ds((base + c) * CH, CH)],
                             sidx.at[slot], sem_i.at[slot])
            pltpu.async_copy(ei1_hbm.at[1, pl.ds((base + c) * CH, CH)],
                             didx.at[slot], sem_i.at[slot])

        @pl.when(cid == 1)
        def _():
            pltpu.async_copy(ei2_hbm.at[0, pl.ds((base + c) * CH, CH)],
                             sidx.at[slot], sem_i.at[slot])
            pltpu.async_copy(ei2_hbm.at[1, pl.ds((base + c) * CH, CH)],
                             didx.at[slot], sem_i.at[slot])

    def _wait_idx_and_offset(slot):
        for _ in range(2):
            pltpu.make_async_copy(ei1_hbm.at[0, pl.ds(0, CH)], sidx.at[slot],
                                  sem_i.at[slot]).wait()
        off = cid * N
        for i in range(CH // 16):
            sidx[slot, pl.ds(16 * i, 16)] = (
                sidx[slot, pl.ds(16 * i, 16)] + off)

    def _gather(c, slot_x, slot_r):
        pltpu.async_copy(h_hbm.at[sidx.at[slot_x]], rows_v.at[slot_r],
                         sem_g.at[slot_r])

    def _wait_gather(slot_x, slot_r):
        pltpu.make_async_copy(h_hbm.at[sidx.at[slot_x]], rows_v.at[slot_r],
                              sem_g.at[slot_r]).wait()

    def _wait_scatter(slot_r, slot_x):
        pltpu.make_async_copy(rows_v.at[slot_r], acc_sh.at[didx.at[slot_x]],
                              sem_s.at[slot_r]).wait()

    _load_idx(0, 0)
    _load_idx(1, 1)
    plsc.subcore_barrier()
    _wait_idx_and_offset(0)
    _gather(0, 0, 0)

    def _chunk(c, carry):
        s_g = lax.rem(c, 3)
        s_x = lax.rem(c, 4)
        c1 = c + 1
        c2 = c + 2
        s1r = lax.rem(c1, 3)
        s1x = lax.rem(c1, 4)

        @pl.when(c2 < nch)
        def _():
            _load_idx(c2, lax.rem(c2, 4))

        @pl.when(c1 < nch)
        def _():
            @pl.when(c >= 2)
            def _():
                _wait_scatter(s1r, lax.rem(c - 2, 4))

            _wait_idx_and_offset(s1x)
            _gather(c1, s1x, s1r)

        _wait_gather(s_x, s_g)
        pltpu.async_copy(rows_v.at[s_g], acc_sh.at[didx.at[s_x]],
                         sem_s.at[s_g], add=True)
        return carry

    lax.fori_loop(0, nch, _chunk, 0)
    for k in range(3):
        c = nch - 3 + k
        _wait_scatter(lax.rem(c, 3), lax.rem(c, 4))
    plsc.subcore_barrier()

    @pl.when(sid < NT - 1)
    def _():
        pltpu.sync_copy(acc_sh.at[pl.ds(sid * RPT, RPT)],
                        out_hbm.at[pl.ds(cid * N + sid * RPT, RPT)])

    @pl.when(sid == NT - 1)
    def _():
        pltpu.sync_copy(acc_sh.at[pl.ds((NT - 1) * RPT, RPT_LAST)],
                        out_hbm.at[pl.ds(cid * N + (NT - 1) * RPT, RPT_LAST)])


@functools.cache
def _agg_kernel_fn(F):
    return pl.kernel(
        _agg_body,
        out_type=jax.ShapeDtypeStruct((2 * N, F), jnp.bfloat16),
        mesh=_get_mesh(),
        scratch_types=[
            pltpu.VMEM((4, CH), jnp.int32),
            pltpu.VMEM((4, CH), jnp.int32),
            pltpu.VMEM((3, CH, F), jnp.bfloat16),
            pltpu.VMEM_SHARED((AGG_ROWS, F), jnp.bfloat16),
            pltpu.SemaphoreType.DMA((4,)),
            pltpu.SemaphoreType.DMA((3,)),
            pltpu.SemaphoreType.DMA((3,)),
        ],
        compiler_params=pltpu.CompilerParams(use_tc_tiling_on_sc=False),
    )


def _agg_kernel(F):
    return _agg_kernel_fn(F)


def _k1a_body(x_ref, w_ref, out_ref):
    out_ref[...] = jnp.dot(x_ref[...], w_ref[...],
                           preferred_element_type=jnp.float32)


def _k1b_body(h_ref, deg_ref, out_ref):
    dinv = lax.rsqrt(deg_ref[...][:, :1])
    out_ref[...] = (h_ref[...] * dinv).astype(jnp.bfloat16)


def _layer_body(acc_ref, deg_ref, b_ref, w_ref, f_ref, h_ref, cs_ref):
    i = pl.program_id(0)
    dinv = lax.rsqrt(deg_ref[...][:, :1])
    f = acc_ref[...].astype(jnp.float32) * dinv + b_ref[...]
    f_ref[...] = f
    r = jnp.maximum(f, 0.0)
    h_ref[...] = (jnp.dot(r, w_ref[...], preferred_element_type=jnp.float32)
                  * dinv).astype(jnp.bfloat16)

    @pl.when(i % (NBLK // 2) == 0)
    def _():
        cs_ref[...] = jnp.zeros_like(cs_ref)

    cs_ref[...] += jnp.sum(f, axis=0)[None, None, :]


def _last_body(acc_ref, deg_ref, b_ref, f_ref, cs_ref):
    i = pl.program_id(0)
    dinv = lax.rsqrt(deg_ref[...][:, :1])
    f = acc_ref[...].astype(jnp.float32) * dinv + b_ref[...]
    f_ref[...] = f

    @pl.when(i % (NBLK // 2) == 0)
    def _():
        cs_ref[...] = jnp.zeros_like(cs_ref)

    cs_ref[...] += jnp.sum(f, axis=0)[None, None, :]


def _attn_body(f_ref, cs_ref, wa_ref, p_ref):
    i = pl.program_id(0)
    gc = jnp.tanh(jnp.dot(cs_ref[0] * (1.0 / N), wa_ref[...],
                          preferred_element_type=jnp.float32))
    f = f_ref[...]
    s = jax.nn.sigmoid(
        lax.dot_general(f, gc, (((1,), (1,)), ((), ())),
                        preferred_element_type=jnp.float32))
    contrib = lax.dot_general(s, f, (((0,), (0,)), ((), ())),
                              preferred_element_type=jnp.float32)

    @pl.when(i % (NBLK // 2) == 0)
    def _():
        p_ref[...] = jnp.zeros_like(p_ref)

    p_ref[...] += contrib[None]


def _ntn_a_body(p1_ref, p2_ref, p3_ref, t1_ref, t2_ref, t3_ref,
                o1_ref, o2_ref, o3_ref):
    for p_ref, t_ref, o_ref in ((p1_ref, t1_ref, o1_ref),
                                (p2_ref, t2_ref, o2_ref),
                                (p3_ref, t3_ref, o3_ref)):
        o_ref[...] = jnp.dot(p_ref[0], t_ref[...],
                             preferred_element_type=jnp.float32)


def _ntn_b_body(m1_ref, m2_ref, m3_ref, p1_ref, p2_ref, p3_ref,
                tb1_ref, tb2_ref, tb3_ref, tc1_ref, tc2_ref, tc3_ref,
                ws1_ref, bs1_ref, ws2_ref, bs2_ref, out_ref):
    parts = []
    for p_ref, m_ref, tbt_ref, tcr_ref in (
            (p1_ref, m1_ref, tb1_ref, tc1_ref),
            (p2_ref, m2_ref, tb2_ref, tc2_ref),
            (p3_ref, m3_ref, tb3_ref, tc3_ref)):
        e1 = p_ref[0]
        e2 = p_ref[1]
        scoring = jnp.dot(e2, m_ref[...], preferred_element_type=jnp.float32)
        comb = jnp.concatenate([e1, e2], axis=1)
        blk = jnp.dot(comb, tbt_ref[...], preferred_element_type=jnp.float32)
        parts.append(jnp.maximum(scoring + blk + tcr_ref[...], 0.0))
    scores = jnp.concatenate(parts, axis=1)
    h = jnp.maximum(jnp.dot(scores, ws1_ref[...],
                            preferred_element_type=jnp.float32) + bs1_ref[...], 0.0)
    out_ref[...] = jax.nn.sigmoid(
        jnp.dot(h, ws2_ref[...], preferred_element_type=jnp.float32) + bs2_ref[...])


def _row_spec(F):
    return pl.BlockSpec((R, F), lambda i: (i, 0))


def _full_spec(shape):
    nd = len(shape)
    return pl.BlockSpec(shape, lambda i, _n=nd: (0,) * _n)


def _cs_spec(F):
    return pl.BlockSpec((1, 1, F), lambda i: (i // (NBLK // 2), 0, 0))


def _tc_k1a(x_st, w1):
    return pl.pallas_call(
        _k1a_body,
        grid=(NBLK,),
        in_specs=[_row_spec(128), _full_spec((128, 128))],
        out_specs=_row_spec(128),
        out_shape=jax.ShapeDtypeStruct((2 * N, 128), jnp.float32),
    )(x_st, w1)


def _tc_k1b(h_raw, deg_st):
    return pl.pallas_call(
        _k1b_body,
        grid=(NBLK,),
        in_specs=[_row_spec(128), _row_spec(16)],
        out_specs=_row_spec(128),
        out_shape=jax.ShapeDtypeStruct((2 * N, 128), jnp.bfloat16),
    )(h_raw, deg_st)


def _tc_layer(acc_st, deg_st, b_row, w_next, Fi, Fo):
    return pl.pallas_call(
        _layer_body,
        grid=(NBLK,),
        in_specs=[_row_spec(Fi), _row_spec(16), _full_spec((1, Fi)),
                  _full_spec((Fi, Fo))],
        out_specs=[_row_spec(Fi), _row_spec(Fo), _cs_spec(Fi)],
        out_shape=[jax.ShapeDtypeStruct((2 * N, Fi), jnp.float32),
                   jax.ShapeDtypeStruct((2 * N, Fo), jnp.bfloat16),
                   jax.ShapeDtypeStruct((NC, 1, Fi), jnp.float32)],
    )(acc_st, deg_st, b_row, w_next)


def _tc_last(acc_st, deg_st, b_row, Fi):
    return pl.pallas_call(
        _last_body,
        grid=(NBLK,),
        in_specs=[_row_spec(Fi), _row_spec(16), _full_spec((1, Fi))],
        out_specs=[_row_spec(Fi), _cs_spec(Fi)],
        out_shape=[jax.ShapeDtypeStruct((2 * N, Fi), jnp.float32),
                   jax.ShapeDtypeStruct((NC, 1, Fi), jnp.float32)],
    )(acc_st, deg_st, b_row)


def _tc_attn1(f, cs, wa, F):
    return pl.pallas_call(
        _attn_body,
        grid=(NBLK,),
        in_specs=[_row_spec(F), _cs_spec(F), _full_spec((F, F))],
        out_specs=_cs_spec(F),
        out_shape=jax.ShapeDtypeStruct((NC, 1, F), jnp.float32),
    )(f, cs, wa)


def _tc_ntn_a(p1, p2, p3, t1f, t2f, t3f):
    return pl.pallas_call(
        _ntn_a_body,
        grid=(1,),
        in_specs=[_full_spec((NC, 1, 128)), _full_spec((NC, 1, 64)),
                  _full_spec((NC, 1, 32)),
                  _full_spec((128, 128 * 128)), _full_spec((64, 64 * 64)),
                  _full_spec((32, 32 * 32))],
        out_specs=[_full_spec((1, 128 * 128)), _full_spec((1, 64 * 64)),
                   _full_spec((1, 32 * 32))],
        out_shape=[jax.ShapeDtypeStruct((1, 128 * 128), jnp.float32),
                   jax.ShapeDtypeStruct((1, 64 * 64), jnp.float32),
                   jax.ShapeDtypeStruct((1, 32 * 32), jnp.float32)],
    )(p1, p2, p3, t1f, t2f, t3f)


def _tc_ntn_b(m1, m2, m3, p1, p2, p3, tb1t, tb2t, tb3t, tc1r, tc2r, tc3r,
              ws1, bs1r, ws2, bs2r):
    return pl.pallas_call(
        _ntn_b_body,
        grid=(1,),
        in_specs=[_full_spec((128, 128)), _full_spec((64, 64)),
                  _full_spec((32, 32)),
                  _full_spec((NC, 1, 128)), _full_spec((NC, 1, 64)),
                  _full_spec((NC, 1, 32)),
                  _full_spec((256, 128)), _full_spec((128, 64)),
                  _full_spec((64, 32)),
                  _full_spec((1, 128)), _full_spec((1, 64)), _full_spec((1, 32)),
                  _full_spec((224, 16)), _full_spec((1, 16)),
                  _full_spec((16, 1)), _full_spec((1, 1))],
        out_specs=_full_spec((1, 1)),
        out_shape=jax.ShapeDtypeStruct((1, 1), jnp.float32),
    )(m1, m2, m3, p1, p2, p3, tb1t, tb2t, tb3t, tc1r, tc2r, tc3r,
      ws1, bs1r, ws2, bs2r)


def kernel(x1, edge_index1, x2, edge_index2, W1, b1, W2, b2, W3, b3,
           Wa1, Wa2, Wa3, T1, Tb1, Tc1, T2, Tb2, Tc2, T3, Tb3, Tc3,
           Ws1, bs1, Ws2, bs2):
    x_st = jnp.concatenate([x1, x2], axis=0)
    b1r, b2r, b3r = b1[None, :], b2[None, :], b3[None, :]
    t1f = T1.reshape(128, 128 * 128)
    t2f = T2.reshape(64, 64 * 64)
    t3f = T3.reshape(32, 32 * 32)
    tb1t, tb2t, tb3t = Tb1.T, Tb2.T, Tb3.T
    tc1r, tc2r, tc3r = Tc1.T, Tc2.T, Tc3.T
    bs1r, bs2r = bs1[None, :], bs2[None, :]

    h1_raw = _tc_k1a(x_st, W1)
    deg_st = _deg_kernel(edge_index1, edge_index2)

    h1 = _tc_k1b(h1_raw, deg_st)
    a1 = _agg_kernel(128)(h1, edge_index1, edge_index2)
    f1, h2, cs1 = _tc_layer(a1, deg_st, b1r, W2, 128, 64)
    a2 = _agg_kernel(64)(h2, edge_index1, edge_index2)
    p1 = _tc_attn1(f1, cs1, Wa1, 128)
    f2, h3, cs2 = _tc_layer(a2, deg_st, b2r, W3, 64, 32)
    a3 = _agg_kernel(32)(h3, edge_index1, edge_index2)
    p2 = _tc_attn1(f2, cs2, Wa2, 64)
    f3, cs3 = _tc_last(a3, deg_st, b3r, 32)
    p3 = _tc_attn1(f3, cs3, Wa3, 32)

    o1, o2, o3 = _tc_ntn_a(p1, p2, p3, t1f, t2f, t3f)
    m1 = o1.reshape(128, 128)
    m2 = o2.reshape(64, 64)
    m3 = o3.reshape(32, 32)
    return _tc_ntn_b(m1, m2, m3, p1, p2, p3, tb1t, tb2t, tb3t,
                     tc1r, tc2r, tc3r, Ws1, bs1r, Ws2, bs2r)


# speedup vs baseline: 1.0417x; 1.0417x over previous
import functools

import jax
import jax.numpy as jnp
from jax import lax
from jax.experimental import pallas as pl
from jax.experimental.pallas import tpu as pltpu
from jax.experimental.pallas import tpu_sc as plsc

N = 10000
E = 320000
NT = 16
NC = 2
CH = 128
NCHB = (E // CH) // NT
XTRA = (E // CH) - NCHB * NT
AGG_ROWS = 10000
DEG_ROWS = 10240
RPT = 640
RPT_LAST = N - (NT - 1) * RPT
R = 2000
NBLK = (2 * N) // R

@functools.cache
def _get_mesh():
    return plsc.VectorSubcoreMesh(core_axis_name="c", subcore_axis_name="s",
                                  num_cores=NC, num_subcores=NT)


def _deg_body(ei1_hbm, ei2_hbm, out_hbm, ones_v, didx, acc_sh, sem_i):
    cid = lax.axis_index("c")
    sid = lax.axis_index("s")
    nch = NCHB + jnp.where(sid < XTRA, 1, 0)
    base = sid * NCHB + jnp.minimum(sid, XTRA)

    def _fill(i, carry):
        ones_v[i] = jnp.ones((16,), jnp.float32)
        return carry

    lax.fori_loop(0, CH, _fill, 0)
    for k in range(DEG_ROWS // NT // CH):
        pltpu.sync_copy(ones_v, acc_sh.at[pl.ds(sid * (DEG_ROWS // NT) + k * CH, CH)])

    def _load_idx(c, slot):
        @pl.when(cid == 0)
        def _():
            pltpu.async_copy(ei1_hbm.at[1, pl.ds((base + c) * CH, CH)],
                             didx.at[slot], sem_i.at[slot])

        @pl.when(cid == 1)
        def _():
            pltpu.async_copy(ei2_hbm.at[1, pl.ds((base + c) * CH, CH)],
                             didx.at[slot], sem_i.at[slot])

    def _wait_idx(c, slot):
        pltpu.make_async_copy(ei1_hbm.at[1, pl.ds(0, CH)], didx.at[slot],
                              sem_i.at[slot]).wait()

    _load_idx(0, 0)
    _load_idx(1, 1)
    _load_idx(2, 2)
    plsc.subcore_barrier()

    def _chunk(c, carry):
        @pl.when(c + 3 < nch)
        def _():
            _load_idx(c + 3, lax.rem(c + 3, 4))

        _wait_idx(c, lax.rem(c, 4))
        pltpu.sync_copy(ones_v, acc_sh.at[didx.at[lax.rem(c, 4)]], add=True)
        return carry

    lax.fori_loop(0, nch, _chunk, 0)
    plsc.subcore_barrier()

@functools.cache
def _deg_kernel_fn():
    return pl.kernel(
        _deg_body,
        out_type=jax.ShapeDtypeStruct((2 * N, 16), jnp.float32),
        mesh=_get_mesh(),
        scratch_types=[
            pltpu.VMEM((CH, 16), jnp.float32),
            pltpu.VMEM((4, CH), jnp.int32),
            pltpu.VMEM_SHARED((DEG_ROWS, 16), jnp.float32),
            pltpu.SemaphoreType.DMA((4,)),
        ],
        compiler_params=pltpu.CompilerParams(use_tc_tiling_on_sc=False),
    )


def _deg_kernel(ei1, ei2):
    return _deg_kernel_fn()(ei1, ei2)


def _agg_body(h_hbm, ei1_hbm, ei2_hbm, out_hbm,
              sidx, didx, rows_v, acc_sh, sem_i, sem_g, sem_s):
    cid = lax.axis_index("c")
    sid = lax.axis_index("s")
    nch = NCHB + jnp.where(sid < XTRA, 1, 0)
    base = sid * NCHB + jnp.minimum(sid, XTRA)

    @pl.when(sid < NT - 1)
    def _():
        pltpu.sync_copy(h_hbm.at[pl.ds(cid * N + sid * RPT, RPT)],
                        acc_sh.at[pl.ds(sid * RPT, RPT)])

    @pl.when(sid == NT - 1)
    def _():
        pltpu.sync_copy(h_hbm.at[pl.ds(cid * N + (NT - 1) * RPT, RPT_LAST)],
                        acc_sh.at[pl.ds((NT - 1) * RPT, RPT_LAST)])

    def _load_idx(c, slot):
        @pl.when(cid == 0)
        def _():
            pltpu.async_copy(ei1_hbm.at[0, pl.ds((base + c) * CH, CH)],
                             sidx.at[slot], sem_i.at[slot])
            pltpu.async_copy(ei1_hbm.at[1, pl.ds((base + c) * CH, CH)],
                             didx.at[slot], sem_i.at[slot])

        @pl.when(cid == 1)
        def _():
            pltpu.async_copy(ei2_hbm.at[0, pl.ds((base + c) * CH, CH)],
                             sidx.at[slot], sem_i.at[slot])
            pltpu.async_copy(ei2_hbm.at[1, pl.ds((base + c) * CH, CH)],
                             didx.at[slot], sem_i.at[slot])

    def _wait_idx_and_offset(slot):
        for _ in range(2):
            pltpu.make_async_copy(ei1_hbm.at[0, pl.ds(0, CH)], sidx.at[slot],
                                  sem_i.at[slot]).wait()
        off = cid * N
        for i in range(CH // 16):
            sidx[slot, pl.ds(16 * i, 16)] = (
                sidx[slot, pl.ds(16 * i, 16)] + off)

    def _gather(c, slot_x, slot_r):
        pltpu.async_copy(h_hbm.at[sidx.at[slot_x]], rows_v.at[slot_r],
                         sem_g.at[slot_r])

    def _wait_gather(slot_x, slot_r):
        pltpu.make_async_copy(h_hbm.at[sidx.at[slot_x]], rows_v.at[slot_r],
                              sem_g.at[slot_r]).wait()

    def _wait_scatter(slot_r, slot_x):
        pltpu.make_async_copy(rows_v.at[slot_r], acc_sh.at[didx.at[slot_x]],
                              sem_s.at[slot_r]).wait()

    _load_idx(0, 0)
    _load_idx(1, 1)
    _load_idx(2, 2)
    plsc.subcore_barrier()
    _wait_idx_and_offset(0)
    _gather(0, 0, 0)

    def _chunk(c, carry):
        s_g = lax.rem(c, 3)
        s_x = lax.rem(c, 5)
        c1 = c + 1
        c3 = c + 3
        s1r = lax.rem(c1, 3)
        s1x = lax.rem(c1, 5)

        @pl.when(c1 < nch)
        def _():
            @pl.when(c >= 2)
            def _():
                _wait_scatter(s1r, lax.rem(c - 2, 5))

            @pl.when(c3 < nch)
            def _():
                _load_idx(c3, lax.rem(c3, 5))

            _wait_idx_and_offset(s1x)
            _gather(c1, s1x, s1r)

        _wait_gather(s_x, s_g)
        pltpu.async_copy(rows_v.at[s_g], acc_sh.at[didx.at[s_x]],
                         sem_s.at[s_g], add=True)
        return carry

    lax.fori_loop(0, nch, _chunk, 0)
    for k in range(3):
        c = nch - 3 + k
        _wait_scatter(lax.rem(c, 3), lax.rem(c, 5))
    plsc.subcore_barrier()

    @pl.when(sid < NT - 1)
    def _():
        pltpu.sync_copy(acc_sh.at[pl.ds(sid * RPT, RPT)],
                        out_hbm.at[pl.ds(cid * N + sid * RPT, RPT)])

    @pl.when(sid == NT - 1)
    def _():
        pltpu.sync_copy(acc_sh.at[pl.ds((NT - 1) * RPT, RPT_LAST)],
                        out_hbm.at[pl.ds(cid * N + (NT - 1) * RPT, RPT_LAST)])


@functools.cache
def _agg_kernel_fn(F):
    return pl.kernel(
        _agg_body,
        out_type=jax.ShapeDtypeStruct((2 * N, F), jnp.bfloat16),
        mesh=_get_mesh(),
        scratch_types=[
            pltpu.VMEM((5, CH), jnp.int32),
            pltpu.VMEM((5, CH), jnp.int32),
            pltpu.VMEM((3, CH, F), jnp.bfloat16),
            pltpu.VMEM_SHARED((AGG_ROWS, F), jnp.bfloat16),
            pltpu.SemaphoreType.DMA((5,)),
            pltpu.SemaphoreType.DMA((3,)),
            pltpu.SemaphoreType.DMA((3,)),
        ],
        compiler_params=pltpu.CompilerParams(use_tc_tiling_on_sc=False),
    )


def _agg_kernel(F):
    return _agg_kernel_fn(F)


def _k1a_body(x_ref, w_ref, out_ref):
    out_ref[...] = jnp.dot(x_ref[...], w_ref[...],
                           preferred_element_type=jnp.float32)


def _k1b_body(h_ref, deg_ref, out_ref):
    dinv = lax.rsqrt(deg_ref[...][:, :1])
    out_ref[...] = (h_ref[...] * dinv).astype(jnp.bfloat16)


def _layer_body(acc_ref, deg_ref, b_ref, w_ref, f_ref, h_ref, cs_ref):
    i = pl.program_id(0)
    dinv = lax.rsqrt(deg_ref[...][:, :1])
    f = acc_ref[...].astype(jnp.float32) * dinv + b_ref[...]
    f_ref[...] = f
    r = jnp.maximum(f, 0.0)
    h_ref[...] = (jnp.dot(r, w_ref[...], preferred_element_type=jnp.float32)
                  * dinv).astype(jnp.bfloat16)

    @pl.when(i % (NBLK // 2) == 0)
    def _():
        cs_ref[...] = jnp.zeros_like(cs_ref)

    cs_ref[...] += jnp.sum(f, axis=0)[None, None, :]


def _last_body(acc_ref, deg_ref, b_ref, f_ref, cs_ref):
    i = pl.program_id(0)
    dinv = lax.rsqrt(deg_ref[...][:, :1])
    f = acc_ref[...].astype(jnp.float32) * dinv + b_ref[...]
    f_ref[...] = f

    @pl.when(i % (NBLK // 2) == 0)
    def _():
        cs_ref[...] = jnp.zeros_like(cs_ref)

    cs_ref[...] += jnp.sum(f, axis=0)[None, None, :]


def _attn_body(f_ref, cs_ref, wa_ref, p_ref):
    i = pl.program_id(0)
    gc = jnp.tanh(jnp.dot(cs_ref[0] * (1.0 / N), wa_ref[...],
                          preferred_element_type=jnp.float32))
    f = f_ref[...]
    s = jax.nn.sigmoid(
        lax.dot_general(f, gc, (((1,), (1,)), ((), ())),
                        preferred_element_type=jnp.float32))
    contrib = lax.dot_general(s, f, (((0,), (0,)), ((), ())),
                              preferred_element_type=jnp.float32)

    @pl.when(i % (NBLK // 2) == 0)
    def _():
        p_ref[...] = jnp.zeros_like(p_ref)

    p_ref[...] += contrib[None]


def _ntn_a_body(p1_ref, p2_ref, p3_ref, t1_ref, t2_ref, t3_ref,
                o1_ref, o2_ref, o3_ref):
    for p_ref, t_ref, o_ref in ((p1_ref, t1_ref, o1_ref),
                                (p2_ref, t2_ref, o2_ref),
                                (p3_ref, t3_ref, o3_ref)):
        o_ref[...] = jnp.dot(p_ref[0], t_ref[...],
                             preferred_element_type=jnp.float32)


def _ntn_b_body(m1_ref, m2_ref, m3_ref, p1_ref, p2_ref, p3_ref,
                tb1_ref, tb2_ref, tb3_ref, tc1_ref, tc2_ref, tc3_ref,
                ws1_ref, bs1_ref, ws2_ref, bs2_ref, out_ref):
    parts = []
    for p_ref, m_ref, tbt_ref, tcr_ref in (
            (p1_ref, m1_ref, tb1_ref, tc1_ref),
            (p2_ref, m2_ref, tb2_ref, tc2_ref),
            (p3_ref, m3_ref, tb3_ref, tc3_ref)):
        e1 = p_ref[0]
        e2 = p_ref[1]
        scoring = jnp.dot(e2, m_ref[...], preferred_element_type=jnp.float32)
        comb = jnp.concatenate([e1, e2], axis=1)
        blk = jnp.dot(comb, tbt_ref[...], preferred_element_type=jnp.float32)
        parts.append(jnp.maximum(scoring + blk + tcr_ref[...], 0.0))
    scores = jnp.concatenate(parts, axis=1)
    h = jnp.maximum(jnp.dot(scores, ws1_ref[...],
                            preferred_element_type=jnp.float32) + bs1_ref[...], 0.0)
    out_ref[...] = jax.nn.sigmoid(
        jnp.dot(h, ws2_ref[...], preferred_element_type=jnp.float32) + bs2_ref[...])


def _row_spec(F):
    return pl.BlockSpec((R, F), lambda i: (i, 0))


def _full_spec(shape):
    nd = len(shape)
    return pl.BlockSpec(shape, lambda i, _n=nd: (0,) * _n)


def _cs_spec(F):
    return pl.BlockSpec((1, 1, F), lambda i: (i // (NBLK // 2), 0, 0))


def _tc_k1a(x_st, w1):
    return pl.pallas_call(
        _k1a_body,
        grid=(NBLK,),
        in_specs=[_row_spec(128), _full_spec((128, 128))],
        out_specs=_row_spec(128),
        out_shape=jax.ShapeDtypeStruct((2 * N, 128), jnp.float32),
    )(x_st, w1)


def _tc_k1b(h_raw, deg_st):
    return pl.pallas_call(
        _k1b_body,
        grid=(NBLK,),
        in_specs=[_row_spec(128), _row_spec(16)],
        out_specs=_row_spec(128),
        out_shape=jax.ShapeDtypeStruct((2 * N, 128), jnp.bfloat16),
    )(h_raw, deg_st)


def _tc_layer(acc_st, deg_st, b_row, w_next, Fi, Fo):
    return pl.pallas_call(
        _layer_body,
        grid=(NBLK,),
        in_specs=[_row_spec(Fi), _row_spec(16), _full_spec((1, Fi)),
                  _full_spec((Fi, Fo))],
        out_specs=[_row_spec(Fi), _row_spec(Fo), _cs_spec(Fi)],
        out_shape=[jax.ShapeDtypeStruct((2 * N, Fi), jnp.float32),
                   jax.ShapeDtypeStruct((2 * N, Fo), jnp.bfloat16),
                   jax.ShapeDtypeStruct((NC, 1, Fi), jnp.float32)],
    )(acc_st, deg_st, b_row, w_next)


def _tc_last(acc_st, deg_st, b_row, Fi):
    return pl.pallas_call(
        _last_body,
        grid=(NBLK,),
        in_specs=[_row_spec(Fi), _row_spec(16), _full_spec((1, Fi))],
        out_specs=[_row_spec(Fi), _cs_spec(Fi)],
        out_shape=[jax.ShapeDtypeStruct((2 * N, Fi), jnp.float32),
                   jax.ShapeDtypeStruct((NC, 1, Fi), jnp.float32)],
    )(acc_st, deg_st, b_row)


def _tc_attn1(f, cs, wa, F):
    return pl.pallas_call(
        _attn_body,
        grid=(NBLK,),
        in_specs=[_row_spec(F), _cs_spec(F), _full_spec((F, F))],
        out_specs=_cs_spec(F),
        out_shape=jax.ShapeDtypeStruct((NC, 1, F), jnp.float32),
    )(f, cs, wa)


def _tc_ntn_a(p1, p2, p3, t1f, t2f, t3f):
    return pl.pallas_call(
        _ntn_a_body,
        grid=(1,),
        in_specs=[_full_spec((NC, 1, 128)), _full_spec((NC, 1, 64)),
                  _full_spec((NC, 1, 32)),
                  _full_spec((128, 128 * 128)), _full_spec((64, 64 * 64)),
                  _full_spec((32, 32 * 32))],
        out_specs=[_full_spec((1, 128 * 128)), _full_spec((1, 64 * 64)),
                   _full_spec((1, 32 * 32))],
        out_shape=[jax.ShapeDtypeStruct((1, 128 * 128), jnp.float32),
                   jax.ShapeDtypeStruct((1, 64 * 64), jnp.float32),
                   jax.ShapeDtypeStruct((1, 32 * 32), jnp.float32)],
    )(p1, p2, p3, t1f, t2f, t3f)


def _tc_ntn_b(m1, m2, m3, p1, p2, p3, tb1t, tb2t, tb3t, tc1r, tc2r, tc3r,
              ws1, bs1r, ws2, bs2r):
    return pl.pallas_call(
        _ntn_b_body,
        grid=(1,),
        in_specs=[_full_spec((128, 128)), _full_spec((64, 64)),
                  _full_spec((32, 32)),
                  _full_spec((NC, 1, 128)), _full_spec((NC, 1, 64)),
                  _full_spec((NC, 1, 32)),
                  _full_spec((256, 128)), _full_spec((128, 64)),
                  _full_spec((64, 32)),
                  _full_spec((1, 128)), _full_spec((1, 64)), _full_spec((1, 32)),
                  _full_spec((224, 16)), _full_spec((1, 16)),
                  _full_spec((16, 1)), _full_spec((1, 1))],
        out_specs=_full_spec((1, 1)),
        out_shape=jax.ShapeDtypeStruct((1, 1), jnp.float32),
    )(m1, m2, m3, p1, p2, p3, tb1t, tb2t, tb3t, tc1r, tc2r, tc3r,
      ws1, bs1r, ws2, bs2r)


def kernel(x1, edge_index1, x2, edge_index2, W1, b1, W2, b2, W3, b3,
           Wa1, Wa2, Wa3, T1, Tb1, Tc1, T2, Tb2, Tc2, T3, Tb3, Tc3,
           Ws1, bs1, Ws2, bs2):
    x_st = jnp.concatenate([x1, x2], axis=0)
    b1r, b2r, b3r = b1[None, :], b2[None, :], b3[None, :]
    t1f = T1.reshape(128, 128 * 128)
    t2f = T2.reshape(64, 64 * 64)
    t3f = T3.reshape(32, 32 * 32)
    tb1t, tb2t, tb3t = Tb1.T, Tb2.T, Tb3.T
    tc1r, tc2r, tc3r = Tc1.T, Tc2.T, Tc3.T
    bs1r, bs2r = bs1[None, :], bs2[None, :]

    h1_raw = _tc_k1a(x_st, W1)
    deg_st = _deg_kernel(edge_index1, edge_index2)

    h1 = _tc_k1b(h1_raw, deg_st)
    a1 = _agg_kernel(128)(h1, edge_index1, edge_index2)
    f1, h2, cs1 = _tc_layer(a1, deg_st, b1r, W2, 128, 64)
    a2 = _agg_kernel(64)(h2, edge_index1, edge_index2)
    p1 = _tc_attn1(f1, cs1, Wa1, 128)
    f2, h3, cs2 = _tc_layer(a2, deg_st, b2r, W3, 64, 32)
    a3 = _agg_kernel(32)(h3, edge_index1, edge_index2)
    p2 = _tc_attn1(f2, cs2, Wa2, 64)
    f3, cs3 = _tc_last(a3, deg_st, b3r, 32)
    p3 = _tc_attn1(f3, cs3, Wa3, 32)

    o1, o2, o3 = _tc_ntn_a(p1, p2, p3, t1f, t2f, t3f)
    m1 = o1.reshape(128, 128)
    m2 = o2.reshape(64, 64)
    m3 = o3.reshape(32, 32)
    return _tc_ntn_b(m1, m2, m3, p1, p2, p3, tb1t, tb2t, tb3t,
                     tc1r, tc2r, tc3r, Ws1, bs1r, Ws2, bs2r)


# speedup vs baseline: 1.0480x; 1.0061x over previous
import functools

import jax
import jax.numpy as jnp
from jax import lax
from jax.experimental import pallas as pl
from jax.experimental.pallas import tpu as pltpu
from jax.experimental.pallas import tpu_sc as plsc

N = 10000
E = 320000
NT = 16
NC = 2
CH = 128
NCHB = (E // CH) // NT
XTRA = (E // CH) - NCHB * NT
AGG_ROWS = 10000
DEG_ROWS = 10240
RPT = 640
RPT_LAST = N - (NT - 1) * RPT
R = 2000
NBLK = (2 * N) // R

@functools.cache
def _get_mesh():
    return plsc.VectorSubcoreMesh(core_axis_name="c", subcore_axis_name="s",
                                  num_cores=NC, num_subcores=NT)


def _deg_body(ei1_hbm, ei2_hbm, out_hbm, ones_v, didx, acc_sh, sem_i):
    cid = lax.axis_index("c")
    sid = lax.axis_index("s")
    nch = NCHB + jnp.where(sid < XTRA, 1, 0)
    base = sid * NCHB + jnp.minimum(sid, XTRA)

    def _fill(i, carry):
        ones_v[i] = jnp.ones((16,), jnp.float32)
        return carry

    lax.fori_loop(0, CH, _fill, 0)
    for k in range(DEG_ROWS // NT // CH):
        pltpu.sync_copy(ones_v, acc_sh.at[pl.ds(sid * (DEG_ROWS // NT) + k * CH, CH)])

    def _load_idx(c, slot):
        @pl.when(cid == 0)
        def _():
            pltpu.async_copy(ei1_hbm.at[1, pl.ds((base + c) * CH, CH)],
                             didx.at[slot], sem_i.at[slot])

        @pl.when(cid == 1)
        def _():
            pltpu.async_copy(ei2_hbm.at[1, pl.ds((base + c) * CH, CH)],
                             didx.at[slot], sem_i.at[slot])

    def _wait_idx(c, slot):
        pltpu.make_async_copy(ei1_hbm.at[1, pl.ds(0, CH)], didx.at[slot],
                              sem_i.at[slot]).wait()

    _load_idx(0, 0)
    _load_idx(1, 1)
    _load_idx(2, 2)
    plsc.subcore_barrier()

    def _chunk(c, carry):
        @pl.when(c + 3 < nch)
        def _():
            _load_idx(c + 3, lax.rem(c + 3, 4))

        _wait_idx(c, lax.rem(c, 4))
        pltpu.sync_copy(ones_v, acc_sh.at[didx.at[lax.rem(c, 4)]], add=True)
        return carry

    lax.fori_loop(0, nch, _chunk, 0)
    plsc.subcore_barrier()

@functools.cache
def _deg_kernel_fn():
    return pl.kernel(
        _deg_body,
        out_type=jax.ShapeDtypeStruct((2 * N, 16), jnp.float32),
        mesh=_get_mesh(),
        scratch_types=[
            pltpu.VMEM((CH, 16), jnp.float32),
            pltpu.VMEM((4, CH), jnp.int32),
            pltpu.VMEM_SHARED((DEG_ROWS, 16), jnp.float32),
            pltpu.SemaphoreType.DMA((4,)),
        ],
        compiler_params=pltpu.CompilerParams(use_tc_tiling_on_sc=False),
    )


def _deg_kernel(ei1, ei2):
    return _deg_kernel_fn()(ei1, ei2)


def _agg_body(h_hbm, ei1_hbm, ei2_hbm, out_hbm,
              sidx, didx, rows_v, acc_sh, sem_i, sem_g, sem_s):
    cid = lax.axis_index("c")
    sid = lax.axis_index("s")
    nch = NCHB + jnp.where(sid < XTRA, 1, 0)
    base = sid * NCHB + jnp.minimum(sid, XTRA)

    @pl.when(sid < NT - 1)
    def _():
        pltpu.sync_copy(h_hbm.at[pl.ds(cid * N + sid * RPT, RPT)],
                        acc_sh.at[pl.ds(sid * RPT, RPT)])

    @pl.when(sid == NT - 1)
    def _():
        pltpu.sync_copy(h_hbm.at[pl.ds(cid * N + (NT - 1) * RPT, RPT_LAST)],
                        acc_sh.at[pl.ds((NT - 1) * RPT, RPT_LAST)])

    def _load_idx(c, slot):
        @pl.when(cid == 0)
        def _():
            pltpu.async_copy(ei1_hbm.at[0, pl.ds((base + c) * CH, CH)],
                             sidx.at[slot], sem_i.at[slot])
            pltpu.async_copy(ei1_hbm.at[1, pl.ds((base + c) * CH, CH)],
                             didx.at[slot], sem_i.at[slot])

        @pl.when(cid == 1)
        def _():
            pltpu.async_copy(ei2_hbm.at[0, pl.ds((base + c) * CH, CH)],
                             sidx.at[slot], sem_i.at[slot])
            pltpu.async_copy(ei2_hbm.at[1, pl.ds((base + c) * CH, CH)],
                             didx.at[slot], sem_i.at[slot])

    def _wait_idx_and_offset(slot):
        for _ in range(2):
            pltpu.make_async_copy(ei1_hbm.at[0, pl.ds(0, CH)], sidx.at[slot],
                                  sem_i.at[slot]).wait()
        off = cid * N
        for i in range(CH // 16):
            sidx[slot, pl.ds(16 * i, 16)] = (
                sidx[slot, pl.ds(16 * i, 16)] + off)

    def _gather(c, slot_x, slot_r):
        pltpu.async_copy(h_hbm.at[sidx.at[slot_x]], rows_v.at[slot_r],
                         sem_g.at[slot_r])

    def _wait_gather(slot_x, slot_r):
        pltpu.make_async_copy(h_hbm.at[sidx.at[slot_x]], rows_v.at[slot_r],
                              sem_g.at[slot_r]).wait()

    def _wait_scatter(slot_r, slot_x):
        pltpu.make_async_copy(rows_v.at[slot_r], acc_sh.at[didx.at[slot_x]],
                              sem_s.at[slot_r]).wait()

    _load_idx(0, 0)
    _load_idx(1, 1)
    _load_idx(2, 2)
    plsc.subcore_barrier()
    _wait_idx_and_offset(0)
    _gather(0, 0, 0)

    def _chunk(c, carry):
        s_g = lax.rem(c, 3)
        s_x = lax.rem(c, 5)
        c1 = c + 1
        c3 = c + 3
        s1r = lax.rem(c1, 3)
        s1x = lax.rem(c1, 5)

        @pl.when(c1 < nch)
        def _():
            @pl.when(c >= 2)
            def _():
                _wait_scatter(s1r, lax.rem(c - 2, 5))

            @pl.when(c3 < nch)
            def _():
                _load_idx(c3, lax.rem(c3, 5))

            _wait_idx_and_offset(s1x)
            _gather(c1, s1x, s1r)

        _wait_gather(s_x, s_g)
        pltpu.async_copy(rows_v.at[s_g], acc_sh.at[didx.at[s_x]],
                         sem_s.at[s_g], add=True)
        return carry

    lax.fori_loop(0, nch, _chunk, 0)
    for k in range(3):
        c = nch - 3 + k
        _wait_scatter(lax.rem(c, 3), lax.rem(c, 5))
    plsc.subcore_barrier()

    @pl.when(sid < NT - 1)
    def _():
        pltpu.sync_copy(acc_sh.at[pl.ds(sid * RPT, RPT)],
                        out_hbm.at[pl.ds(cid * N + sid * RPT, RPT)])

    @pl.when(sid == NT - 1)
    def _():
        pltpu.sync_copy(acc_sh.at[pl.ds((NT - 1) * RPT, RPT_LAST)],
                        out_hbm.at[pl.ds(cid * N + (NT - 1) * RPT, RPT_LAST)])


@functools.cache
def _agg_kernel_fn(F):
    return pl.kernel(
        _agg_body,
        out_type=jax.ShapeDtypeStruct((2 * N, F), jnp.bfloat16),
        mesh=_get_mesh(),
        scratch_types=[
            pltpu.VMEM((5, CH), jnp.int32),
            pltpu.VMEM((5, CH), jnp.int32),
            pltpu.VMEM((3, CH, F), jnp.bfloat16),
            pltpu.VMEM_SHARED((AGG_ROWS, F), jnp.bfloat16),
            pltpu.SemaphoreType.DMA((5,)),
            pltpu.SemaphoreType.DMA((3,)),
            pltpu.SemaphoreType.DMA((3,)),
        ],
        compiler_params=pltpu.CompilerParams(use_tc_tiling_on_sc=False),
    )


def _agg_kernel(F):
    return _agg_kernel_fn(F)


def _k1a_body(x_ref, w_ref, out_ref):
    out_ref[...] = jnp.dot(x_ref[...], w_ref[...],
                           preferred_element_type=jnp.float32)


def _k1b_body(h_ref, deg_ref, out_ref):
    dinv = lax.rsqrt(deg_ref[...][:, :1])
    out_ref[...] = (h_ref[...] * dinv).astype(jnp.bfloat16)


def _layer_body(acc_ref, deg_ref, b_ref, w_ref, f_ref, h_ref, cs_ref):
    i = pl.program_id(0)
    dinv = lax.rsqrt(deg_ref[...][:, :1])
    f = acc_ref[...].astype(jnp.float32) * dinv + b_ref[...]
    f_ref[...] = f
    r = jnp.maximum(f, 0.0)
    h_ref[...] = (jnp.dot(r, w_ref[...], preferred_element_type=jnp.float32)
                  * dinv).astype(jnp.bfloat16)

    @pl.when(i % (NBLK // 2) == 0)
    def _():
        cs_ref[...] = jnp.zeros_like(cs_ref)

    cs_ref[...] += jnp.sum(f, axis=0)[None, None, :]


def _last_body(acc_ref, deg_ref, b_ref, wa_ref, p_ref, f_buf, cs_buf):
    i = pl.program_id(0)
    half = NBLK // 2

    @pl.when(i < NBLK)
    def _():
        g = i // half
        dinv = lax.rsqrt(deg_ref[...][:, :1])
        f = acc_ref[...].astype(jnp.float32) * dinv + b_ref[...]
        f_buf[pl.ds(i * R, R), :] = f

        @pl.when(i % half == 0)
        def _():
            cs_buf[pl.ds(g, 1)] = jnp.zeros((1, 32), jnp.float32)

        cs_buf[pl.ds(g, 1)] += jnp.sum(f, axis=0)[None, :]

    @pl.when(i >= NBLK)
    def _():
        i2 = i - NBLK
        g = i2 // half
        cs = cs_buf[pl.ds(g, 1)]
        gc = jnp.tanh(jnp.dot(cs * (1.0 / N), wa_ref[...],
                              preferred_element_type=jnp.float32))
        f = f_buf[pl.ds(i2 * R, R), :]
        sg = jax.nn.sigmoid(
            lax.dot_general(f, gc, (((1,), (1,)), ((), ())),
                            preferred_element_type=jnp.float32))
        contrib = lax.dot_general(sg, f, (((0,), (0,)), ((), ())),
                                  preferred_element_type=jnp.float32)

        @pl.when(i2 % half == 0)
        def _():
            p_ref[...] = jnp.zeros_like(p_ref)

        p_ref[...] += contrib[None]


def _attn_body(f_ref, cs_ref, wa_ref, p_ref):
    i = pl.program_id(0)
    gc = jnp.tanh(jnp.dot(cs_ref[0] * (1.0 / N), wa_ref[...],
                          preferred_element_type=jnp.float32))
    f = f_ref[...]
    s = jax.nn.sigmoid(
        lax.dot_general(f, gc, (((1,), (1,)), ((), ())),
                        preferred_element_type=jnp.float32))
    contrib = lax.dot_general(s, f, (((0,), (0,)), ((), ())),
                              preferred_element_type=jnp.float32)

    @pl.when(i % (NBLK // 2) == 0)
    def _():
        p_ref[...] = jnp.zeros_like(p_ref)

    p_ref[...] += contrib[None]


def _ntn_a_body(p1_ref, p2_ref, p3_ref, t1_ref, t2_ref, t3_ref,
                o1_ref, o2_ref, o3_ref):
    for p_ref, t_ref, o_ref in ((p1_ref, t1_ref, o1_ref),
                                (p2_ref, t2_ref, o2_ref),
                                (p3_ref, t3_ref, o3_ref)):
        o_ref[...] = jnp.dot(p_ref[0], t_ref[...],
                             preferred_element_type=jnp.float32)


def _ntn_b_body(m1_ref, m2_ref, m3_ref, p1_ref, p2_ref, p3_ref,
                tb1_ref, tb2_ref, tb3_ref, tc1_ref, tc2_ref, tc3_ref,
                ws1_ref, bs1_ref, ws2_ref, bs2_ref, out_ref):
    parts = []
    for p_ref, m_ref, tbt_ref, tcr_ref in (
            (p1_ref, m1_ref, tb1_ref, tc1_ref),
            (p2_ref, m2_ref, tb2_ref, tc2_ref),
            (p3_ref, m3_ref, tb3_ref, tc3_ref)):
        e1 = p_ref[0]
        e2 = p_ref[1]
        scoring = jnp.dot(e2, m_ref[...], preferred_element_type=jnp.float32)
        comb = jnp.concatenate([e1, e2], axis=1)
        blk = jnp.dot(comb, tbt_ref[...], preferred_element_type=jnp.float32)
        parts.append(jnp.maximum(scoring + blk + tcr_ref[...], 0.0))
    scores = jnp.concatenate(parts, axis=1)
    h = jnp.maximum(jnp.dot(scores, ws1_ref[...],
                            preferred_element_type=jnp.float32) + bs1_ref[...], 0.0)
    out_ref[...] = jax.nn.sigmoid(
        jnp.dot(h, ws2_ref[...], preferred_element_type=jnp.float32) + bs2_ref[...])


def _row_spec(F):
    return pl.BlockSpec((R, F), lambda i: (i, 0))


def _full_spec(shape):
    nd = len(shape)
    return pl.BlockSpec(shape, lambda i, _n=nd: (0,) * _n)


def _cs_spec(F):
    return pl.BlockSpec((1, 1, F), lambda i: (i // (NBLK // 2), 0, 0))


def _tc_k1a(x_st, w1):
    return pl.pallas_call(
        _k1a_body,
        grid=(NBLK,),
        in_specs=[_row_spec(128), _full_spec((128, 128))],
        out_specs=_row_spec(128),
        out_shape=jax.ShapeDtypeStruct((2 * N, 128), jnp.float32),
    )(x_st, w1)


def _tc_k1b(h_raw, deg_st):
    return pl.pallas_call(
        _k1b_body,
        grid=(NBLK,),
        in_specs=[_row_spec(128), _row_spec(16)],
        out_specs=_row_spec(128),
        out_shape=jax.ShapeDtypeStruct((2 * N, 128), jnp.bfloat16),
    )(h_raw, deg_st)


def _tc_layer(acc_st, deg_st, b_row, w_next, Fi, Fo):
    return pl.pallas_call(
        _layer_body,
        grid=(NBLK,),
        in_specs=[_row_spec(Fi), _row_spec(16), _full_spec((1, Fi)),
                  _full_spec((Fi, Fo))],
        out_specs=[_row_spec(Fi), _row_spec(Fo), _cs_spec(Fi)],
        out_shape=[jax.ShapeDtypeStruct((2 * N, Fi), jnp.float32),
                   jax.ShapeDtypeStruct((2 * N, Fo), jnp.bfloat16),
                   jax.ShapeDtypeStruct((NC, 1, Fi), jnp.float32)],
    )(acc_st, deg_st, b_row, w_next)


def _tc_last(acc_st, deg_st, b_row, wa):
    half = NBLK // 2
    clamp = lambda i: jnp.maximum(i - NBLK, 0) if hasattr(i, "dtype") else max(i - NBLK, 0)
    return pl.pallas_call(
        _last_body,
        grid=(2 * NBLK,),
        in_specs=[
            pl.BlockSpec((R, 32), lambda i: (i % NBLK, 0)),
            pl.BlockSpec((R, 16), lambda i: (i % NBLK, 0)),
            _full_spec((1, 32)),
            _full_spec((32, 32)),
        ],
        out_specs=pl.BlockSpec(
            (1, 1, 32), lambda i: (jnp.maximum(i - NBLK, 0) // (NBLK // 2), 0, 0)),
        out_shape=jax.ShapeDtypeStruct((NC, 1, 32), jnp.float32),
        scratch_shapes=[
            pltpu.VMEM((2 * N, 32), jnp.float32),
            pltpu.VMEM((NC, 32), jnp.float32),
        ],
    )(acc_st, deg_st, b_row, wa)


def _tc_attn1(f, cs, wa, F):
    return pl.pallas_call(
        _attn_body,
        grid=(NBLK,),
        in_specs=[_row_spec(F), _cs_spec(F), _full_spec((F, F))],
        out_specs=_cs_spec(F),
        out_shape=jax.ShapeDtypeStruct((NC, 1, F), jnp.float32),
    )(f, cs, wa)


def _tc_ntn_a(p1, p2, p3, t1f, t2f, t3f):
    return pl.pallas_call(
        _ntn_a_body,
        grid=(1,),
        in_specs=[_full_spec((NC, 1, 128)), _full_spec((NC, 1, 64)),
                  _full_spec((NC, 1, 32)),
                  _full_spec((128, 128 * 128)), _full_spec((64, 64 * 64)),
                  _full_spec((32, 32 * 32))],
        out_specs=[_full_spec((1, 128 * 128)), _full_spec((1, 64 * 64)),
                   _full_spec((1, 32 * 32))],
        out_shape=[jax.ShapeDtypeStruct((1, 128 * 128), jnp.float32),
                   jax.ShapeDtypeStruct((1, 64 * 64), jnp.float32),
                   jax.ShapeDtypeStruct((1, 32 * 32), jnp.float32)],
    )(p1, p2, p3, t1f, t2f, t3f)


def _tc_ntn_b(m1, m2, m3, p1, p2, p3, tb1t, tb2t, tb3t, tc1r, tc2r, tc3r,
              ws1, bs1r, ws2, bs2r):
    return pl.pallas_call(
        _ntn_b_body,
        grid=(1,),
        in_specs=[_full_spec((128, 128)), _full_spec((64, 64)),
                  _full_spec((32, 32)),
                  _full_spec((NC, 1, 128)), _full_spec((NC, 1, 64)),
                  _full_spec((NC, 1, 32)),
                  _full_spec((256, 128)), _full_spec((128, 64)),
                  _full_spec((64, 32)),
                  _full_spec((1, 128)), _full_spec((1, 64)), _full_spec((1, 32)),
                  _full_spec((224, 16)), _full_spec((1, 16)),
                  _full_spec((16, 1)), _full_spec((1, 1))],
        out_specs=_full_spec((1, 1)),
        out_shape=jax.ShapeDtypeStruct((1, 1), jnp.float32),
    )(m1, m2, m3, p1, p2, p3, tb1t, tb2t, tb3t, tc1r, tc2r, tc3r,
      ws1, bs1r, ws2, bs2r)


def kernel(x1, edge_index1, x2, edge_index2, W1, b1, W2, b2, W3, b3,
           Wa1, Wa2, Wa3, T1, Tb1, Tc1, T2, Tb2, Tc2, T3, Tb3, Tc3,
           Ws1, bs1, Ws2, bs2):
    x_st = jnp.concatenate([x1, x2], axis=0)
    b1r, b2r, b3r = b1[None, :], b2[None, :], b3[None, :]
    t1f = T1.reshape(128, 128 * 128)
    t2f = T2.reshape(64, 64 * 64)
    t3f = T3.reshape(32, 32 * 32)
    tb1t, tb2t, tb3t = Tb1.T, Tb2.T, Tb3.T
    tc1r, tc2r, tc3r = Tc1.T, Tc2.T, Tc3.T
    bs1r, bs2r = bs1[None, :], bs2[None, :]

    h1_raw = _tc_k1a(x_st, W1)
    deg_st = _deg_kernel(edge_index1, edge_index2)

    h1 = _tc_k1b(h1_raw, deg_st)
    a1 = _agg_kernel(128)(h1, edge_index1, edge_index2)
    f1, h2, cs1 = _tc_layer(a1, deg_st, b1r, W2, 128, 64)
    a2 = _agg_kernel(64)(h2, edge_index1, edge_index2)
    p1 = _tc_attn1(f1, cs1, Wa1, 128)
    f2, h3, cs2 = _tc_layer(a2, deg_st, b2r, W3, 64, 32)
    a3 = _agg_kernel(32)(h3, edge_index1, edge_index2)
    p2 = _tc_attn1(f2, cs2, Wa2, 64)
    p3 = _tc_last(a3, deg_st, b3r, Wa3)

    o1, o2, o3 = _tc_ntn_a(p1, p2, p3, t1f, t2f, t3f)
    m1 = o1.reshape(128, 128)
    m2 = o2.reshape(64, 64)
    m3 = o3.reshape(32, 32)
    return _tc_ntn_b(m1, m2, m3, p1, p2, p3, tb1t, tb2t, tb3t,
                     tc1r, tc2r, tc3r, Ws1, bs1r, Ws2, bs2r)
